# baseline (speedup 1.0000x reference)
import functools

import jax
import jax.numpy as jnp
from jax import lax
from jax.experimental import pallas as pl
from jax.experimental.pallas import tpu as pltpu
from jax.experimental.pallas import tpu_sc as plsc

D = 64
B = 16384
SEG = 154

TILES_OFF = 0
PORTS_OFF = 1368
STRUCT_OFF = 1424
ROADS_OFF = 1856
TBL_ROWS = 14336

ROWS_TOTAL = B * SEG
NW = 32
ROWS_PER_W = ROWS_TOTAL // NW
CHUNK = 128
CHUNKS_PER_W = ROWS_PER_W // CHUNK
IDX_BLK = 56
N_BLKS = CHUNKS_PER_W // IDX_BLK
TBL_ROWS_PER_TILE = TBL_ROWS // 16


def _onehot_dot(idx_col, n, tbl):
  rows = idx_col.shape[0]
  oh = (idx_col == lax.broadcasted_iota(jnp.int32, (rows, n), 1))
  return jnp.dot(oh.astype(jnp.float32), tbl,
                 preferred_element_type=jnp.float32)


def _tbl_kernel(tt, res, dice, pos, pres, ppos, own, styp, npos, out):
  i = lax.broadcasted_iota(jnp.int32, (1368, 1), 0)
  r, rem = i // 228, i % 228
  d, p = rem // 19, rem % 19
  out[0:1368, :] = (_onehot_dot(r, 6, res[...]) +
                    _onehot_dot(d, 12, dice[...]) +
                    _onehot_dot(p, 19, pos[...]) + tt[0:1, :])
  i = lax.broadcasted_iota(jnp.int32, (56, 1), 0)
  pr, pp = i // 9, i % 9
  out[pl.ds(PORTS_OFF, 56), :] = (_onehot_dot(pr, 6, pres[...]) +
                                  _onehot_dot(pp, 9, ppos[...]) + tt[1:2, :])
  i = lax.broadcasted_iota(jnp.int32, (432, 1), 0)
  o, rem = i // 108, i % 108
  t, p = rem // 54, rem % 54
  out[pl.ds(STRUCT_OFF, 432), :] = (_onehot_dot(o, 4, own[...]) +
                                    _onehot_dot(t, 2, styp[...]) +
                                    _onehot_dot(p, 54, npos[...]) + tt[2:3, :])
  i = lax.broadcasted_iota(jnp.int32, (11664, 1), 0)
  o, rem = i // 2916, i % 2916
  a, b = rem // 54, rem % 54
  out[pl.ds(ROADS_OFF, 11664), :] = (_onehot_dot(o, 4, own[...]) +
                                     _onehot_dot(a, 54, npos[...]) +
                                     _onehot_dot(b, 54, npos[...]) + tt[3:4, :])
  out[pl.ds(13520, TBL_ROWS - 13520), :] = jnp.zeros(
      (TBL_ROWS - 13520, D), jnp.float32)


def _build_tbl(tt, res, dice, pos, pres, ppos, own, styp, npos):
  return pl.pallas_call(
      _tbl_kernel,
      out_shape=jax.ShapeDtypeStruct((TBL_ROWS, D), jnp.float32),
  )(tt, res, dice, pos, pres, ppos, own, styp, npos)


def _idx_kernel(tr, td, tp, pr, pp, so, st, sp, ro, ra, rb, out):
  tiles = tr[...] * 228 + td[...] * 19 + tp[...]
  ports = PORTS_OFF + pr[...] * 9 + pp[...]
  strct = STRUCT_OFF + so[...] * 108 + st[...] * 54 + sp[...]
  roads = ROADS_OFF + ro[...] * 2916 + ra[...] * 54 + rb[...]
  out[...] = jnp.concatenate([tiles, ports, strct, roads], axis=1)


def _build_idx(tr, td, tp, pr, pp, so, st, sp, ro, ra, rb):
  bs = 2048
  grid = B // bs

  def spec(w):
    return pl.BlockSpec((bs, w), lambda i: (i, 0))

  return pl.pallas_call(
      _idx_kernel,
      grid=(grid,),
      in_specs=[spec(19), spec(19), spec(19), spec(9), spec(9),
                spec(54), spec(54), spec(54), spec(72), spec(72), spec(72)],
      out_specs=spec(SEG),
      out_shape=jax.ShapeDtypeStruct((B, SEG), jnp.int32),
  )(tr, td, tp, pr, pp, so, st, sp, ro, ra, rb)


@functools.lru_cache(maxsize=None)
def _make_sc_gather():
  @functools.partial(
      pl.kernel,
      mesh=plsc.VectorSubcoreMesh(core_axis_name="c", subcore_axis_name="s",
                                  num_cores=2, num_subcores=16),
      out_type=jax.ShapeDtypeStruct((ROWS_TOTAL, D), jnp.float32),
      compiler_params=pltpu.CompilerParams(use_tc_tiling_on_sc=False),
      scratch_types=[
          pltpu.VMEM((IDX_BLK, CHUNK), jnp.int32),
          pltpu.VMEM((CHUNK, D), jnp.float32),
          pltpu.SemaphoreType.DMA,
      ],
  )
  def _sc_gather(idx_hbm, tbl_hbm, out_hbm, idx_v, rows_v, gsem):
    c = lax.axis_index("c")
    s = lax.axis_index("s")
    wid = s * 2 + c

    base = wid * ROWS_PER_W
    cbase = wid * CHUNKS_PER_W

    def blk(b, carry):
      pltpu.sync_copy(idx_hbm.at[pl.ds(cbase + b * IDX_BLK, IDX_BLK)], idx_v)

      def chunk(g, carry2):
        pltpu.async_copy(tbl_hbm.at[idx_v.at[g]], rows_v, gsem).wait()
        pltpu.sync_copy(
            rows_v,
            out_hbm.at[pl.ds(base + (b * IDX_BLK + g) * CHUNK, CHUNK)])
        return carry2

      lax.fori_loop(0, IDX_BLK, chunk, 0)
      return carry

    lax.fori_loop(0, N_BLKS, blk, 0)

  return _sc_gather


FLAT_ROWS = ROWS_TOTAL * D // 128
RELAYOUT_BS = 64
FLAT_PER_BS = RELAYOUT_BS * SEG * D // 128


def _relayout_kernel(flat, out):
  x = flat[...].reshape(RELAYOUT_BS, SEG // 2, 128)
  lo = x[:, :, 0:64][:, :, None, :]
  hi = x[:, :, 64:128][:, :, None, :]
  out[...] = jnp.concatenate([lo, hi], axis=2).reshape(RELAYOUT_BS, SEG, D)


def _relayout(flat):
  return pl.pallas_call(
      _relayout_kernel,
      grid=(B // RELAYOUT_BS,),
      in_specs=[pl.BlockSpec((FLAT_PER_BS, 128), lambda i: (i, 0))],
      out_specs=pl.BlockSpec((RELAYOUT_BS, SEG, D), lambda i: (i, 0, 0)),
      out_shape=jax.ShapeDtypeStruct((B, SEG, D), jnp.float32),
  )(flat)


def kernel(tile_resource, tile_dicenum, tile_pos, port_resource, port_pos,
           struct_owner, struct_type, struct_pos, road_owner, road_a, road_b,
           tiletype_embed, resource_embed, dicenum_embed, position_embed,
           port_resource_embed, port_position_embed, owner_embed,
           structure_type_embed, node_pos_embed):
  tbl = _build_tbl(tiletype_embed, resource_embed, dicenum_embed,
                   position_embed, port_resource_embed, port_position_embed,
                   owner_embed, structure_type_embed, node_pos_embed)
  idx = _build_idx(tile_resource.astype(jnp.int32),
                   tile_dicenum.astype(jnp.int32),
                   tile_pos.astype(jnp.int32),
                   port_resource.astype(jnp.int32),
                   port_pos.astype(jnp.int32),
                   struct_owner.astype(jnp.int32),
                   struct_type.astype(jnp.int32),
                   struct_pos.astype(jnp.int32),
                   road_owner.astype(jnp.int32),
                   road_a.astype(jnp.int32),
                   road_b.astype(jnp.int32))
  idx2 = idx.reshape(ROWS_TOTAL // CHUNK, CHUNK)
  out = _make_sc_gather()(idx2, tbl)
  return _relayout(out.reshape(FLAT_ROWS, 128))


# speedup vs baseline: 1.4219x; 1.4219x over previous
import functools

import jax
import jax.numpy as jnp
from jax import lax
from jax.experimental import pallas as pl
from jax.experimental.pallas import tpu as pltpu
from jax.experimental.pallas import tpu_sc as plsc

D = 64
B = 16384
SEG = 154

TILES_OFF = 0
PORTS_OFF = 1368
STRUCT_OFF = 1424
ROADS_OFF = 1856
TBL_ROWS = 14336

ROWS_TOTAL = B * SEG
NW = 32
ROWS_PER_W = ROWS_TOTAL // NW
CHUNK = 128
CHUNKS_PER_W = ROWS_PER_W // CHUNK
IDX_BLK = 56
N_BLKS = CHUNKS_PER_W // IDX_BLK
TBL_ROWS_PER_TILE = TBL_ROWS // 16


def _onehot_dot(idx_col, n, tbl):
  rows = idx_col.shape[0]
  oh = (idx_col == lax.broadcasted_iota(jnp.int32, (rows, n), 1))
  return jnp.dot(oh.astype(jnp.float32), tbl,
                 preferred_element_type=jnp.float32)


def _tbl_kernel(tt, res, dice, pos, pres, ppos, own, styp, npos, out):
  i = lax.broadcasted_iota(jnp.int32, (1368, 1), 0)
  r, rem = i // 228, i % 228
  d, p = rem // 19, rem % 19
  out[0:1368, :] = (_onehot_dot(r, 6, res[...]) +
                    _onehot_dot(d, 12, dice[...]) +
                    _onehot_dot(p, 19, pos[...]) + tt[0:1, :])
  i = lax.broadcasted_iota(jnp.int32, (56, 1), 0)
  pr, pp = i // 9, i % 9
  out[pl.ds(PORTS_OFF, 56), :] = (_onehot_dot(pr, 6, pres[...]) +
                                  _onehot_dot(pp, 9, ppos[...]) + tt[1:2, :])
  i = lax.broadcasted_iota(jnp.int32, (432, 1), 0)
  o, rem = i // 108, i % 108
  t, p = rem // 54, rem % 54
  out[pl.ds(STRUCT_OFF, 432), :] = (_onehot_dot(o, 4, own[...]) +
                                    _onehot_dot(t, 2, styp[...]) +
                                    _onehot_dot(p, 54, npos[...]) + tt[2:3, :])
  i = lax.broadcasted_iota(jnp.int32, (11664, 1), 0)
  o, rem = i // 2916, i % 2916
  a, b = rem // 54, rem % 54
  out[pl.ds(ROADS_OFF, 11664), :] = (_onehot_dot(o, 4, own[...]) +
                                     _onehot_dot(a, 54, npos[...]) +
                                     _onehot_dot(b, 54, npos[...]) + tt[3:4, :])
  out[pl.ds(13520, TBL_ROWS - 13520), :] = jnp.zeros(
      (TBL_ROWS - 13520, D), jnp.float32)


def _build_tbl(tt, res, dice, pos, pres, ppos, own, styp, npos):
  return pl.pallas_call(
      _tbl_kernel,
      out_shape=jax.ShapeDtypeStruct((TBL_ROWS, D), jnp.float32),
  )(tt, res, dice, pos, pres, ppos, own, styp, npos)


def _idx_kernel(tr, td, tp, pr, pp, so, st, sp, ro, ra, rb, out):
  tiles = tr[...] * 228 + td[...] * 19 + tp[...]
  ports = PORTS_OFF + pr[...] * 9 + pp[...]
  strct = STRUCT_OFF + so[...] * 108 + st[...] * 54 + sp[...]
  roads = ROADS_OFF + ro[...] * 2916 + ra[...] * 54 + rb[...]
  out[...] = jnp.concatenate([tiles, ports, strct, roads], axis=1)


def _build_idx(tr, td, tp, pr, pp, so, st, sp, ro, ra, rb):
  bs = 2048
  grid = B // bs

  def spec(w):
    return pl.BlockSpec((bs, w), lambda i: (i, 0))

  return pl.pallas_call(
      _idx_kernel,
      grid=(grid,),
      in_specs=[spec(19), spec(19), spec(19), spec(9), spec(9),
                spec(54), spec(54), spec(54), spec(72), spec(72), spec(72)],
      out_specs=spec(SEG),
      out_shape=jax.ShapeDtypeStruct((B, SEG), jnp.int32),
  )(tr, td, tp, pr, pp, so, st, sp, ro, ra, rb)


@functools.lru_cache(maxsize=None)
def _make_sc_gather():
  @functools.partial(
      pl.kernel,
      mesh=plsc.VectorSubcoreMesh(core_axis_name="c", subcore_axis_name="s",
                                  num_cores=2, num_subcores=16),
      out_type=jax.ShapeDtypeStruct((ROWS_TOTAL, D), jnp.float32),
      compiler_params=pltpu.CompilerParams(use_tc_tiling_on_sc=False),
      scratch_types=[
          pltpu.VMEM((CHUNKS_PER_W, CHUNK), jnp.int32),
          pltpu.VMEM((2, CHUNK, D), jnp.float32),
          pltpu.SemaphoreType.DMA,
          pltpu.SemaphoreType.DMA,
      ],
  )
  def _sc_gather(idx_hbm, tbl_hbm, out_hbm, idx_v, rows_v, gsem, wsem):
    c = lax.axis_index("c")
    s = lax.axis_index("s")
    wid = s * 2 + c

    base = wid * ROWS_PER_W
    cbase = wid * CHUNKS_PER_W

    pltpu.sync_copy(idx_hbm.at[pl.ds(cbase, CHUNKS_PER_W)], idx_v)

    def gather(g, buf):
      return pltpu.make_async_copy(tbl_hbm.at[idx_v.at[g]], rows_v.at[buf],
                                   gsem)

    def write(g, buf):
      return pltpu.make_async_copy(
          rows_v.at[buf], out_hbm.at[pl.ds(base + g * CHUNK, CHUNK)], wsem)

    gather(0, 0).start()

    def chunk(g, carry):
      buf = lax.rem(g, 2)
      @pl.when(g >= 2)
      def _():
        write(g - 2, buf).wait()

      gather(g, buf).wait()

      @pl.when(g + 1 < CHUNKS_PER_W)
      def _():
        gather(g + 1, 1 - buf).start()

      write(g, buf).start()
      return carry

    lax.fori_loop(0, CHUNKS_PER_W, chunk, 0)
    write(CHUNKS_PER_W - 2, 0).wait()
    write(CHUNKS_PER_W - 1, 1).wait()

  return _sc_gather


def kernel(tile_resource, tile_dicenum, tile_pos, port_resource, port_pos,
           struct_owner, struct_type, struct_pos, road_owner, road_a, road_b,
           tiletype_embed, resource_embed, dicenum_embed, position_embed,
           port_resource_embed, port_position_embed, owner_embed,
           structure_type_embed, node_pos_embed):
  tbl = _build_tbl(tiletype_embed, resource_embed, dicenum_embed,
                   position_embed, port_resource_embed, port_position_embed,
                   owner_embed, structure_type_embed, node_pos_embed)
  idx = _build_idx(tile_resource.astype(jnp.int32),
                   tile_dicenum.astype(jnp.int32),
                   tile_pos.astype(jnp.int32),
                   port_resource.astype(jnp.int32),
                   port_pos.astype(jnp.int32),
                   struct_owner.astype(jnp.int32),
                   struct_type.astype(jnp.int32),
                   struct_pos.astype(jnp.int32),
                   road_owner.astype(jnp.int32),
                   road_a.astype(jnp.int32),
                   road_b.astype(jnp.int32))
  idx2 = idx.reshape(ROWS_TOTAL // CHUNK, CHUNK)
  out = _make_sc_gather()(idx2, tbl)
  return out.reshape(B, SEG, D)


# speedup vs baseline: 2.1354x; 1.5018x over previous
import functools

import jax
import jax.numpy as jnp
from jax import lax
from jax.experimental import pallas as pl
from jax.experimental.pallas import tpu as pltpu
from jax.experimental.pallas import tpu_sc as plsc

D = 64
B = 16384
SEG = 154

TILES_OFF = 0
PORTS_OFF = 1368
STRUCT_OFF = 1424
ROADS_OFF = 1856
TBL_ROWS = 13520


def _onehot_dot(idx_col, n, tbl):
  rows = idx_col.shape[0]
  oh = (idx_col == lax.broadcasted_iota(jnp.int32, (rows, n), 1))
  return jnp.dot(oh.astype(jnp.float32), tbl,
                 preferred_element_type=jnp.float32)


def _tbl_kernel(tt, res, dice, pos, pres, ppos, own, styp, npos, out):
  i = lax.broadcasted_iota(jnp.int32, (1368, 1), 0)
  r, rem = i // 228, i % 228
  d, p = rem // 19, rem % 19
  out[0:1368, :] = (_onehot_dot(r, 6, res[...]) +
                    _onehot_dot(d, 12, dice[...]) +
                    _onehot_dot(p, 19, pos[...]) + tt[0:1, :])
  i = lax.broadcasted_iota(jnp.int32, (56, 1), 0)
  pr, pp = i // 9, i % 9
  out[pl.ds(PORTS_OFF, 56), :] = (_onehot_dot(pr, 6, pres[...]) +
                                  _onehot_dot(pp, 9, ppos[...]) + tt[1:2, :])
  i = lax.broadcasted_iota(jnp.int32, (432, 1), 0)
  o, rem = i // 108, i % 108
  t, p = rem // 54, rem % 54
  out[pl.ds(STRUCT_OFF, 432), :] = (_onehot_dot(o, 4, own[...]) +
                                    _onehot_dot(t, 2, styp[...]) +
                                    _onehot_dot(p, 54, npos[...]) + tt[2:3, :])
  i = lax.broadcasted_iota(jnp.int32, (11664, 1), 0)
  o, rem = i // 2916, i % 2916
  a, b = rem // 54, rem % 54
  out[pl.ds(ROADS_OFF, 11664), :] = (_onehot_dot(o, 4, own[...]) +
                                     _onehot_dot(a, 54, npos[...]) +
                                     _onehot_dot(b, 54, npos[...]) + tt[3:4, :])


def _build_tbl(tt, res, dice, pos, pres, ppos, own, styp, npos):
  return pl.pallas_call(
      _tbl_kernel,
      out_shape=jax.ShapeDtypeStruct((TBL_ROWS, D), jnp.float32),
  )(tt, res, dice, pos, pres, ppos, own, styp, npos)


def _idx_kernel(tr, td, tp, pr, pp, so, st, sp, ro, ra, rb, out):
  tiles = tr[...] * 228 + td[...] * 19 + tp[...]
  ports = PORTS_OFF + pr[...] * 9 + pp[...]
  strct = STRUCT_OFF + so[...] * 108 + st[...] * 54 + sp[...]
  roads = ROADS_OFF + ro[...] * 2916 + ra[...] * 54 + rb[...]
  out[...] = jnp.concatenate([tiles, ports, strct, roads], axis=1)


def _build_idx(tr, td, tp, pr, pp, so, st, sp, ro, ra, rb):
  bs = 2048
  grid = B // bs

  def spec(w):
    return pl.BlockSpec((bs, w), lambda i: (i, 0))

  return pl.pallas_call(
      _idx_kernel,
      grid=(grid,),
      in_specs=[spec(19), spec(19), spec(19), spec(9), spec(9),
                spec(54), spec(54), spec(54), spec(72), spec(72), spec(72)],
      out_specs=spec(SEG),
      out_shape=jax.ShapeDtypeStruct((B, SEG), jnp.int32),
  )(tr, td, tp, pr, pp, so, st, sp, ro, ra, rb)


M_OUT = SEG * D * B // 128
B_GRP = B // 4
D_TILES = 8


@functools.lru_cache(maxsize=None)
def _make_sc_gather():
  @functools.partial(
      pl.kernel,
      mesh=plsc.VectorSubcoreMesh(core_axis_name="c", subcore_axis_name="s",
                                  num_cores=2, num_subcores=16),
      out_type=jax.ShapeDtypeStruct((M_OUT, 128), jnp.float32),
      compiler_params=pltpu.CompilerParams(use_tc_tiling_on_sc=False,
                                           needs_layout_passes=False),
      scratch_types=[
          pltpu.VMEM((8, TBL_ROWS), jnp.float32),
          pltpu.VMEM((2, B_GRP), jnp.int32),
          pltpu.VMEM((2, 16, 128), jnp.float32),
          pltpu.SemaphoreType.DMA,
          pltpu.SemaphoreType.DMA,
      ],
  )
  def _sc_gather(idx_hbm, tblt_hbm, out_hbm, tbl_v, idx_v, stage_v,
                 isem, wsem):
    c = lax.axis_index("c")
    sax = lax.axis_index("s")
    dt = lax.rem(sax, D_TILES)
    grp = (sax // D_TILES) * 2 + c

    pltpu.sync_copy(tblt_hbm.at[pl.ds(dt * 8, 8)], tbl_v)

    def idx_load(s_seg, buf):
      return pltpu.make_async_copy(
          idx_hbm.at[s_seg, pl.ds(grp * B_GRP, B_GRP)], idx_v.at[buf], isem)

    def wait_write():
      pltpu.make_async_copy(stage_v.at[0], out_hbm.at[pl.ds(0, 16)],
                            wsem).wait()

    idx_load(0, 0).start()

    def s_body(s_seg, carry):
      sbuf = lax.rem(s_seg, 2)
      idx_load(s_seg, sbuf).wait()

      @pl.when(s_seg + 1 < SEG)
      def _():
        idx_load(s_seg + 1, 1 - sbuf).start()

      mrow = s_seg * 8192 + dt * 1024 + grp * 256

      def u_body(u, carry2):
        ubuf = lax.rem(u, 2)

        @pl.when((s_seg > 0) | (u >= 2))
        def _():
          wait_write()

        def l_body(l, carry3):
          off = u * 256 + l * 16
          iv0 = idx_v[sbuf, pl.ds(off, 16)]
          iv1 = idx_v[sbuf, pl.ds(off + 128, 16)]
          for sub in range(8):
            srow = jnp.full((16,), sub, jnp.int32)
            stage_v[ubuf, sub, pl.ds(l * 16, 16)] = plsc.load_gather(
                tbl_v, [srow, iv0])
            stage_v[ubuf, 8 + sub, pl.ds(l * 16, 16)] = plsc.load_gather(
                tbl_v, [srow, iv1])
          return carry3

        lax.fori_loop(0, 8, l_body, 0)
        pltpu.make_async_copy(
            stage_v.at[ubuf], out_hbm.at[pl.ds(mrow + u * 16, 16)],
            wsem).start()
        return carry2

      lax.fori_loop(0, 16, u_body, 0)
      return carry

    lax.fori_loop(0, SEG, s_body, 0)
    wait_write()
    wait_write()

  return _sc_gather


def kernel(tile_resource, tile_dicenum, tile_pos, port_resource, port_pos,
           struct_owner, struct_type, struct_pos, road_owner, road_a, road_b,
           tiletype_embed, resource_embed, dicenum_embed, position_embed,
           port_resource_embed, port_position_embed, owner_embed,
           structure_type_embed, node_pos_embed):
  tbl = _build_tbl(tiletype_embed, resource_embed, dicenum_embed,
                   position_embed, port_resource_embed, port_position_embed,
                   owner_embed, structure_type_embed, node_pos_embed)
  idx = _build_idx(tile_resource.astype(jnp.int32),
                   tile_dicenum.astype(jnp.int32),
                   tile_pos.astype(jnp.int32),
                   port_resource.astype(jnp.int32),
                   port_pos.astype(jnp.int32),
                   struct_owner.astype(jnp.int32),
                   struct_type.astype(jnp.int32),
                   struct_pos.astype(jnp.int32),
                   road_owner.astype(jnp.int32),
                   road_a.astype(jnp.int32),
                   road_b.astype(jnp.int32))
  out2d = _make_sc_gather()(jnp.transpose(idx), jnp.transpose(tbl))
  return (out2d.reshape(SEG, 8, 128, 8, 128)
          .transpose(2, 4, 0, 1, 3)
          .reshape(B, SEG, D))


# speedup vs baseline: 2.1423x; 1.0032x over previous
import functools

import jax
import jax.numpy as jnp
from jax import lax
from jax.experimental import pallas as pl
from jax.experimental.pallas import tpu as pltpu
from jax.experimental.pallas import tpu_sc as plsc

D = 64
B = 16384
SEG = 154

TILES_OFF = 0
PORTS_OFF = 1368
STRUCT_OFF = 1424
ROADS_OFF = 1856
TBL_ROWS = 13520


def _onehot_dot(idx_col, n, tbl):
  rows = idx_col.shape[0]
  oh = (idx_col == lax.broadcasted_iota(jnp.int32, (rows, n), 1))
  return jnp.dot(oh.astype(jnp.float32), tbl,
                 preferred_element_type=jnp.float32)


def _tbl_kernel(tt, res, dice, pos, pres, ppos, own, styp, npos, out):
  i = lax.broadcasted_iota(jnp.int32, (1368, 1), 0)
  r, rem = i // 228, i % 228
  d, p = rem // 19, rem % 19
  out[0:1368, :] = (_onehot_dot(r, 6, res[...]) +
                    _onehot_dot(d, 12, dice[...]) +
                    _onehot_dot(p, 19, pos[...]) + tt[0:1, :])
  i = lax.broadcasted_iota(jnp.int32, (56, 1), 0)
  pr, pp = i // 9, i % 9
  out[pl.ds(PORTS_OFF, 56), :] = (_onehot_dot(pr, 6, pres[...]) +
                                  _onehot_dot(pp, 9, ppos[...]) + tt[1:2, :])
  i = lax.broadcasted_iota(jnp.int32, (432, 1), 0)
  o, rem = i // 108, i % 108
  t, p = rem // 54, rem % 54
  out[pl.ds(STRUCT_OFF, 432), :] = (_onehot_dot(o, 4, own[...]) +
                                    _onehot_dot(t, 2, styp[...]) +
                                    _onehot_dot(p, 54, npos[...]) + tt[2:3, :])
  i = lax.broadcasted_iota(jnp.int32, (11664, 1), 0)
  o, rem = i // 2916, i % 2916
  a, b = rem // 54, rem % 54
  out[pl.ds(ROADS_OFF, 11664), :] = (_onehot_dot(o, 4, own[...]) +
                                     _onehot_dot(a, 54, npos[...]) +
                                     _onehot_dot(b, 54, npos[...]) + tt[3:4, :])


def _build_tbl(tt, res, dice, pos, pres, ppos, own, styp, npos):
  return pl.pallas_call(
      _tbl_kernel,
      out_shape=jax.ShapeDtypeStruct((TBL_ROWS, D), jnp.float32),
  )(tt, res, dice, pos, pres, ppos, own, styp, npos)


def _idx_kernel(tr, td, tp, pr, pp, so, st, sp, ro, ra, rb, out):
  tiles = tr[...] * 228 + td[...] * 19 + tp[...]
  ports = PORTS_OFF + pr[...] * 9 + pp[...]
  strct = STRUCT_OFF + so[...] * 108 + st[...] * 54 + sp[...]
  roads = ROADS_OFF + ro[...] * 2916 + ra[...] * 54 + rb[...]
  out[...] = jnp.concatenate([tiles, ports, strct, roads], axis=1)


def _build_idx(tr, td, tp, pr, pp, so, st, sp, ro, ra, rb):
  bs = 2048
  grid = B // bs

  def spec(w):
    return pl.BlockSpec((bs, w), lambda i: (i, 0))

  return pl.pallas_call(
      _idx_kernel,
      grid=(grid,),
      in_specs=[spec(19), spec(19), spec(19), spec(9), spec(9),
                spec(54), spec(54), spec(54), spec(72), spec(72), spec(72)],
      out_specs=spec(SEG),
      out_shape=jax.ShapeDtypeStruct((B, SEG), jnp.int32),
  )(tr, td, tp, pr, pp, so, st, sp, ro, ra, rb)


M_OUT = SEG * D * B // 128
B_GRP = B // 4
D_TILES = 8


@functools.lru_cache(maxsize=None)
def _make_sc_gather():
  @functools.partial(
      pl.kernel,
      mesh=plsc.VectorSubcoreMesh(core_axis_name="c", subcore_axis_name="s",
                                  num_cores=2, num_subcores=16),
      out_type=jax.ShapeDtypeStruct((M_OUT, 128), jnp.float32),
      compiler_params=pltpu.CompilerParams(use_tc_tiling_on_sc=False,
                                           needs_layout_passes=False),
      scratch_types=[
          pltpu.VMEM((8, TBL_ROWS), jnp.float32),
          pltpu.VMEM((2, B_GRP), jnp.int32),
          pltpu.VMEM((3, 16, 128), jnp.float32),
          pltpu.SemaphoreType.DMA,
          pltpu.SemaphoreType.DMA,
      ],
  )
  def _sc_gather(idx_hbm, tblt_hbm, out_hbm, tbl_v, idx_v, stage_v,
                 isem, wsem):
    c = lax.axis_index("c")
    sax = lax.axis_index("s")
    dt = lax.rem(sax, D_TILES)
    grp = (sax // D_TILES) * 2 + c

    pltpu.sync_copy(tblt_hbm.at[pl.ds(dt * 8, 8)], tbl_v)

    def idx_load(s_seg, buf):
      return pltpu.make_async_copy(
          idx_hbm.at[s_seg, pl.ds(grp * B_GRP, B_GRP)], idx_v.at[buf], isem)

    def wait_write():
      pltpu.make_async_copy(stage_v.at[0], out_hbm.at[pl.ds(0, 16)],
                            wsem).wait()

    idx_load(0, 0).start()
    srows = [jnp.full((16,), sub, jnp.int32) for sub in range(8)]

    def s_body(s_seg, carry):
      sbuf = lax.rem(s_seg, 2)
      idx_load(s_seg, sbuf).wait()

      @pl.when(s_seg + 1 < SEG)
      def _():
        idx_load(s_seg + 1, 1 - sbuf).start()

      mrow = s_seg * 8192 + dt * 1024 + grp * 256

      def u_body(u, carry2):
        w = s_seg * 16 + u
        ubuf = lax.rem(w, 3)

        @pl.when(w >= 3)
        def _():
          wait_write()

        for l in range(8):
          off = u * 256 + l * 16
          iv0 = idx_v[sbuf, pl.ds(off, 16)]
          iv1 = idx_v[sbuf, pl.ds(off + 128, 16)]
          for sub in range(8):
            stage_v[ubuf, sub, pl.ds(l * 16, 16)] = plsc.load_gather(
                tbl_v, [srows[sub], iv0])
            stage_v[ubuf, 8 + sub, pl.ds(l * 16, 16)] = plsc.load_gather(
                tbl_v, [srows[sub], iv1])

        pltpu.make_async_copy(
            stage_v.at[ubuf], out_hbm.at[pl.ds(mrow + u * 16, 16)],
            wsem).start()
        return carry2

      lax.fori_loop(0, 16, u_body, 0)
      return carry

    lax.fori_loop(0, SEG, s_body, 0)
    wait_write()
    wait_write()
    wait_write()

  return _sc_gather


def kernel(tile_resource, tile_dicenum, tile_pos, port_resource, port_pos,
           struct_owner, struct_type, struct_pos, road_owner, road_a, road_b,
           tiletype_embed, resource_embed, dicenum_embed, position_embed,
           port_resource_embed, port_position_embed, owner_embed,
           structure_type_embed, node_pos_embed):
  tbl = _build_tbl(tiletype_embed, resource_embed, dicenum_embed,
                   position_embed, port_resource_embed, port_position_embed,
                   owner_embed, structure_type_embed, node_pos_embed)
  idx = _build_idx(tile_resource.astype(jnp.int32),
                   tile_dicenum.astype(jnp.int32),
                   tile_pos.astype(jnp.int32),
                   port_resource.astype(jnp.int32),
                   port_pos.astype(jnp.int32),
                   struct_owner.astype(jnp.int32),
                   struct_type.astype(jnp.int32),
                   struct_pos.astype(jnp.int32),
                   road_owner.astype(jnp.int32),
                   road_a.astype(jnp.int32),
                   road_b.astype(jnp.int32))
  out2d = _make_sc_gather()(jnp.transpose(idx), jnp.transpose(tbl))
  return (out2d.reshape(SEG, 8, 128, 8, 128)
          .transpose(2, 4, 0, 1, 3)
          .reshape(B, SEG, D))
